# baseline (speedup 1.0000x reference)
import jax
import jax.numpy as jnp
from jax import lax
from jax.experimental import pallas as pl
from jax.experimental.pallas import tpu as pltpu
from jax.experimental.pallas import tpu_sc as plsc

N_IN = 600000
N_OUT = 120000
F = 32
K = 9
R = K * F
NB = 60
BLK_SEG = N_OUT // NB
BLK_ROWS = BLK_SEG * K
TILES = 16
NCORES = 2
BLK_PER_CORE = NB // NCORES
CHUNK = 128
GROUPS = CHUNK // 16
DUMP = BLK_ROWS
SH_ROWS = BLK_ROWS + 16
SLAB = 48
TIN_R = SLAB * K
ROWS_PT = R // TILES


def _sc_body(feat_hbm, dt_hbm, times_hbm, nrate_hbm, kid_hbm, seg_hbm,
             bounds_hbm, zeros_hbm, out_hbm,
             acc1, acc2, times_v, feat_v, dt_v, seg_v, kid_v, vals_v, idx_v,
             bounds_v, nrate_v, tin, tout, sem):
    c = lax.axis_index("c")
    t = lax.axis_index("s")
    pltpu.sync_copy(bounds_hbm, bounds_v)
    pltpu.sync_copy(nrate_hbm, nrate_v)
    nrate_lo = nrate_v[pl.ds(0, 16)]
    nrate_hi = nrate_v[pl.ds(16, 16)]
    iota = lax.broadcasted_iota(jnp.int32, (16,), 0)
    seg0 = jnp.where(t < 10, 128 * t, 1280 + 120 * (t - 10))
    nseg = jnp.where(t < 10, 128, 120)

    def block_body(j, carry):
        b = c * BLK_PER_CORE + j
        base = b * BLK_SEG
        zrow = seg0 * K

        @pl.when(t < 10)
        def _():
            pltpu.sync_copy(zeros_hbm, acc1.at[pl.ds(zrow, 1152)])

        @pl.when(t >= 10)
        def _():
            pltpu.sync_copy(zeros_hbm.at[pl.ds(0, 1080)],
                            acc1.at[pl.ds(zrow, 1080)])

        pltpu.sync_copy(times_hbm.at[pl.ds(base, BLK_SEG)], times_v)
        plsc.subcore_barrier()
        bv = bounds_v[pl.ds(b, 16)]
        lo = bv[0]
        hi = bv[1]
        n = hi - lo
        sh = (n + TILES - 1) // TILES
        a = lo + t * sh
        bb = jnp.minimum(a + sh, hi)
        start0 = (a // 8) * 8
        nc = jnp.maximum((bb - start0 + CHUNK - 1) // CHUNK, 0)

        def chunk_body(ci, carry2):
            cs = jnp.minimum(start0 + ci * CHUNK, N_IN - CHUNK)
            lo_c = jnp.maximum(a, start0 + ci * CHUNK)
            hi_c = jnp.minimum(bb, start0 + ci * CHUNK + CHUNK)
            cp1 = pltpu.async_copy(feat_hbm.at[pl.ds(cs, CHUNK)], feat_v, sem)
            cp2 = pltpu.async_copy(dt_hbm.at[pl.ds(cs, CHUNK)], dt_v, sem)
            cp3 = pltpu.async_copy(seg_hbm.at[pl.ds(cs, CHUNK)], seg_v, sem)
            cp4 = pltpu.async_copy(kid_hbm.at[pl.ds(cs, CHUNK)], kid_v, sem)
            cp1.wait(); cp2.wait(); cp3.wait(); cp4.wait()
            for g in range(GROUPS):
                off = g * 16
                sg = seg_v[pl.ds(off, 16)]
                kd = kid_v[pl.ds(off, 16)]
                dtv = dt_v[pl.ds(off, 16)]
                relc = jnp.clip(sg - base, 0, BLK_SEG - 1)
                tv = plsc.load_gather(times_v, [relc])
                delta = tv - dtv
                gi = cs + off + iota
                valid = (gi >= lo_c) & (gi < hi_c)
                idx = jnp.where(valid, relc * K + kd, DUMP)
                idx_v[pl.ds(off, 16)] = idx
                for i in range(16):
                    d_s = delta[i]
                    e_lo = jnp.exp(d_s * nrate_lo)
                    e_hi = jnp.exp(d_s * nrate_hi)
                    r = off + i
                    vals_v[r, pl.ds(0, 16)] = feat_v[r, pl.ds(0, 16)] * e_lo
                    vals_v[r, pl.ds(16, 16)] = feat_v[r, pl.ds(16, 16)] * e_hi
            pltpu.sync_copy(vals_v, acc1.at[idx_v], add=True)
            return carry2

        lax.fori_loop(0, nc, chunk_body, 0)
        plsc.subcore_barrier()

        def slab_body(sj, cs2):
            sbase = seg0 + jnp.minimum(sj * SLAB, nseg - SLAB)
            pltpu.sync_copy(acc1.at[pl.ds(sbase * K, TIN_R)], tin)
            for k in range(K):
                for f in range(F):
                    colf = jnp.full((16,), f, jnp.int32)
                    for g in range(SLAB // 16):
                        v = plsc.load_gather(
                            tin, [iota * K + (g * 16 * K + k), colf])
                        tout[k * F + f, pl.ds(g * 16, 16)] = v
            pltpu.sync_copy(tout, acc2.at[:, pl.ds(sbase, SLAB)])
            return cs2

        lax.fori_loop(0, 3, slab_body, 0)
        plsc.subcore_barrier()
        for u in range(ROWS_PT):
            r = t * ROWS_PT + u
            pltpu.sync_copy(acc2.at[pl.ds(r, 1)],
                            out_hbm.at[pl.ds(r, 1), pl.ds(base, BLK_SEG)])
        return carry

    lax.fori_loop(0, BLK_PER_CORE, block_body, 0)


def kernel(features, dt, times_out, decay_rate, successor_kernel_ids,
           segment_ids_out):
    nrate = -jax.nn.softplus(decay_rate).astype(jnp.float32)
    starts = (jnp.arange(NB + 1, dtype=jnp.int32) * BLK_SEG)
    bounds = jnp.searchsorted(segment_ids_out, starts,
                              method="compare_all").astype(jnp.int32)
    bounds80 = jnp.concatenate(
        [bounds, jnp.full((80 - (NB + 1),), N_IN, dtype=jnp.int32)])
    zeros_c = jnp.zeros((1152, F), dtype=jnp.float32)

    kern = pl.kernel(
        _sc_body,
        out_type=jax.ShapeDtypeStruct((R, N_OUT), jnp.float32),
        mesh=plsc.VectorSubcoreMesh(core_axis_name="c", subcore_axis_name="s"),
        scratch_types=[
            pltpu.VMEM_SHARED((SH_ROWS, F), jnp.float32),
            pltpu.VMEM_SHARED((R, BLK_SEG), jnp.float32),
            pltpu.VMEM((BLK_SEG,), jnp.float32),
            pltpu.VMEM((CHUNK, F), jnp.float32),
            pltpu.VMEM((CHUNK,), jnp.float32),
            pltpu.VMEM((CHUNK,), jnp.int32),
            pltpu.VMEM((CHUNK,), jnp.int32),
            pltpu.VMEM((CHUNK, F), jnp.float32),
            pltpu.VMEM((CHUNK,), jnp.int32),
            pltpu.VMEM((80,), jnp.int32),
            pltpu.VMEM((F,), jnp.float32),
            pltpu.VMEM((TIN_R, F), jnp.float32),
            pltpu.VMEM((R, SLAB), jnp.float32),
            pltpu.SemaphoreType.DMA,
        ],
        compiler_params=pltpu.CompilerParams(
            needs_layout_passes=False, use_tc_tiling_on_sc=False),
    )
    out2d = kern(features, dt, times_out, nrate, successor_kernel_ids,
                 segment_ids_out, bounds80, zeros_c)
    return out2d.reshape(K, F, N_OUT).transpose(2, 0, 1)


# speedup vs baseline: 1.5856x; 1.5856x over previous
import jax
import jax.numpy as jnp
from jax import lax
from jax.experimental import pallas as pl
from jax.experimental.pallas import tpu as pltpu
from jax.experimental.pallas import tpu_sc as plsc

N_IN = 600000
N_OUT = 120000
F = 32
K = 9
NB = 30
BLK_SEG = N_OUT // NB
BLK_ROWS = BLK_SEG * K
TILES = 16
NCORES = 2
BLK_PER_CORE = NB // NCORES
CH_S = 80
CH_R = CH_S * K
NCH = BLK_ROWS // CH_R
CHUNK = 128
GROUPS = CHUNK // 16
DUMP = BLK_ROWS
SH_ROWS = BLK_ROWS + 16


def _sc_body(feat_hbm, dt_hbm, times_hbm, nrate_hbm, kid_hbm, seg_hbm,
             bounds_hbm, zeros_hbm, out_hbm,
             shared, times_v, feat_v, dt_v, seg_v, kid_v, vals_v, idx_v,
             bounds_v, nrate_v, zbuf, sem):
    c = lax.axis_index("c")
    t = lax.axis_index("s")
    pltpu.sync_copy(bounds_hbm, bounds_v)
    pltpu.sync_copy(nrate_hbm, nrate_v)
    pltpu.sync_copy(zeros_hbm, zbuf)
    nrate_lo = nrate_v[pl.ds(0, 16)]
    nrate_hi = nrate_v[pl.ds(16, 16)]
    iota = lax.broadcasted_iota(jnp.int32, (16,), 0)

    def block_body(j, carry):
        b = c * BLK_PER_CORE + j
        base = b * BLK_SEG
        row0 = b * BLK_ROWS
        nq = (NCH - t + TILES - 1) // TILES

        def zero_body(q2, cz):
            r0 = (t + q2 * TILES) * CH_R
            pltpu.sync_copy(zbuf, shared.at[pl.ds(r0, CH_R)])
            return cz

        lax.fori_loop(0, nq, zero_body, 0)
        pltpu.sync_copy(times_hbm.at[pl.ds(base, BLK_SEG)], times_v)
        plsc.subcore_barrier()
        bv = bounds_v[pl.ds(b, 16)]
        lo = bv[0]
        hi = bv[1]
        n = hi - lo
        sh = (n + TILES - 1) // TILES
        a = lo + t * sh
        bb = jnp.minimum(a + sh, hi)
        start0 = (a // 8) * 8
        nc = jnp.maximum((bb - start0 + CHUNK - 1) // CHUNK, 0)

        def chunk_body(ci, carry2):
            cs = jnp.minimum(start0 + ci * CHUNK, N_IN - CHUNK)
            lo_c = jnp.maximum(a, start0 + ci * CHUNK)
            hi_c = jnp.minimum(bb, start0 + ci * CHUNK + CHUNK)
            cp1 = pltpu.async_copy(feat_hbm.at[pl.ds(cs, CHUNK)], feat_v, sem)
            cp2 = pltpu.async_copy(dt_hbm.at[pl.ds(cs, CHUNK)], dt_v, sem)
            cp3 = pltpu.async_copy(seg_hbm.at[pl.ds(cs, CHUNK)], seg_v, sem)
            cp4 = pltpu.async_copy(kid_hbm.at[pl.ds(cs, CHUNK)], kid_v, sem)
            cp1.wait(); cp2.wait(); cp3.wait(); cp4.wait()
            for g in range(GROUPS):
                off = g * 16
                sg = seg_v[pl.ds(off, 16)]
                kd = kid_v[pl.ds(off, 16)]
                dtv = dt_v[pl.ds(off, 16)]
                relc = jnp.clip(sg - base, 0, BLK_SEG - 1)
                tv = plsc.load_gather(times_v, [relc])
                delta = tv - dtv
                gi = cs + off + iota
                valid = (gi >= lo_c) & (gi < hi_c)
                idx = jnp.where(valid, relc * K + kd, DUMP)
                idx_v[pl.ds(off, 16)] = idx
                for i in range(16):
                    d_s = delta[i]
                    e_lo = jnp.exp(d_s * nrate_lo)
                    e_hi = jnp.exp(d_s * nrate_hi)
                    r = off + i
                    vals_v[r, pl.ds(0, 16)] = feat_v[r, pl.ds(0, 16)] * e_lo
                    vals_v[r, pl.ds(16, 16)] = feat_v[r, pl.ds(16, 16)] * e_hi
            pltpu.sync_copy(vals_v, shared.at[idx_v], add=True)
            return carry2

        lax.fori_loop(0, nc, chunk_body, 0)
        plsc.subcore_barrier()

        def drain_body(q2, cd):
            r0 = (t + q2 * TILES) * CH_R
            pltpu.sync_copy(shared.at[pl.ds(r0, CH_R)],
                            out_hbm.at[pl.ds(row0 + r0, CH_R)])
            return cd

        lax.fori_loop(0, nq, drain_body, 0)
        return carry

    lax.fori_loop(0, BLK_PER_CORE, block_body, 0)


def kernel(features, dt, times_out, decay_rate, successor_kernel_ids,
           segment_ids_out):
    nrate = -jax.nn.softplus(decay_rate).astype(jnp.float32)
    starts = (jnp.arange(NB + 1, dtype=jnp.int32) * BLK_SEG)
    bounds = jnp.searchsorted(segment_ids_out, starts,
                              method="compare_all").astype(jnp.int32)
    bounds48 = jnp.concatenate(
        [bounds, jnp.full((48 - (NB + 1),), N_IN, dtype=jnp.int32)])
    zeros_c = jnp.zeros((CH_R, F), dtype=jnp.float32)

    kern = pl.kernel(
        _sc_body,
        out_type=jax.ShapeDtypeStruct((N_OUT * K, F), jnp.float32),
        mesh=plsc.VectorSubcoreMesh(core_axis_name="c", subcore_axis_name="s"),
        scratch_types=[
            pltpu.VMEM_SHARED((SH_ROWS, F), jnp.float32),
            pltpu.VMEM((BLK_SEG,), jnp.float32),
            pltpu.VMEM((CHUNK, F), jnp.float32),
            pltpu.VMEM((CHUNK,), jnp.float32),
            pltpu.VMEM((CHUNK,), jnp.int32),
            pltpu.VMEM((CHUNK,), jnp.int32),
            pltpu.VMEM((CHUNK, F), jnp.float32),
            pltpu.VMEM((CHUNK,), jnp.int32),
            pltpu.VMEM((48,), jnp.int32),
            pltpu.VMEM((F,), jnp.float32),
            pltpu.VMEM((CH_R, F), jnp.float32),
            pltpu.SemaphoreType.DMA,
        ],
        compiler_params=pltpu.CompilerParams(
            needs_layout_passes=False, use_tc_tiling_on_sc=False),
    )
    out2d = kern(features, dt, times_out, nrate, successor_kernel_ids,
                 segment_ids_out, bounds48, zeros_c)
    return out2d.reshape(N_OUT, K, F)


# speedup vs baseline: 1.6579x; 1.0456x over previous
import jax
import jax.numpy as jnp
from jax import lax
from jax.experimental import pallas as pl
from jax.experimental.pallas import tpu as pltpu
from jax.experimental.pallas import tpu_sc as plsc

N_IN = 600000
N_OUT = 120000
F = 32
K = 9
NB = 30
BLK_SEG = N_OUT // NB
BLK_ROWS = BLK_SEG * K
TILES = 16
NCORES = 2
BLK_PER_CORE = NB // NCORES
CH_S = 80
CH_R = CH_S * K
NCH = BLK_ROWS // CH_R
CHUNK = 128
GROUPS = CHUNK // 16
DUMP = BLK_ROWS
SH_ROWS = BLK_ROWS + 16


def _sc_body(feat_hbm, dt_hbm, times_hbm, nrate_hbm, kid_hbm, seg_hbm,
             bounds_hbm, zeros_hbm, out_hbm,
             shared, times_v, feat_v0, dt_v0, seg_v0, kid_v0, vals_v0,
             idx_v0, feat_v1, dt_v1, seg_v1, kid_v1, vals_v1, idx_v1,
             bounds_v, nrate_v, zbuf, sem):
    feat_vs = (feat_v0, feat_v1)
    dt_vs = (dt_v0, dt_v1)
    seg_vs = (seg_v0, seg_v1)
    kid_vs = (kid_v0, kid_v1)
    vals_vs = (vals_v0, vals_v1)
    idx_vs = (idx_v0, idx_v1)
    c = lax.axis_index("c")
    t = lax.axis_index("s")
    pltpu.sync_copy(bounds_hbm, bounds_v)
    pltpu.sync_copy(nrate_hbm, nrate_v)
    pltpu.sync_copy(zeros_hbm, zbuf)
    nrate_lo = nrate_v[pl.ds(0, 16)]
    nrate_hi = nrate_v[pl.ds(16, 16)]
    iota = lax.broadcasted_iota(jnp.int32, (16,), 0)

    def block_body(j, carry):
        b = c * BLK_PER_CORE + j
        base = b * BLK_SEG
        row0 = b * BLK_ROWS
        nq = (NCH - t + TILES - 1) // TILES

        def zero_body(q2, cz):
            r0 = (t + q2 * TILES) * CH_R
            pltpu.sync_copy(zbuf, shared.at[pl.ds(r0, CH_R)])
            return cz

        lax.fori_loop(0, nq, zero_body, 0)
        pltpu.sync_copy(times_hbm.at[pl.ds(base, BLK_SEG)], times_v)
        plsc.subcore_barrier()
        bv = bounds_v[pl.ds(b, 16)]
        lo = bv[0]
        hi = bv[1]
        n = hi - lo
        sh = (n + TILES - 1) // TILES
        a = lo + t * sh
        bb = jnp.minimum(a + sh, hi)
        start0 = (a // 8) * 8
        nc = jnp.maximum((bb - start0 + CHUNK - 1) // CHUNK, 0)

        nc2 = (nc + 1) // 2

        def pair_body(cp2i, carry2):
            cps = []
            css = []
            for h in range(2):
                ci = cp2i * 2 + h
                cs = jnp.minimum(start0 + ci * CHUNK, N_IN - CHUNK)
                css.append(cs)
                cps.append(pltpu.async_copy(
                    feat_hbm.at[pl.ds(cs, CHUNK)], feat_vs[h], sem))
                cps.append(pltpu.async_copy(
                    dt_hbm.at[pl.ds(cs, CHUNK)], dt_vs[h], sem))
                cps.append(pltpu.async_copy(
                    seg_hbm.at[pl.ds(cs, CHUNK)], seg_vs[h], sem))
                cps.append(pltpu.async_copy(
                    kid_hbm.at[pl.ds(cs, CHUNK)], kid_vs[h], sem))
            scats = []
            for h in range(2):
                ci = cp2i * 2 + h
                cs = css[h]
                lo_c = jnp.maximum(a, start0 + ci * CHUNK)
                hi_c = jnp.minimum(
                    jnp.where(ci < nc, bb, a), start0 + ci * CHUNK + CHUNK)
                for cp in cps[h * 4:h * 4 + 4]:
                    cp.wait()
                for g in range(GROUPS):
                    off = g * 16
                    sg = seg_vs[h][pl.ds(off, 16)]
                    kd = kid_vs[h][pl.ds(off, 16)]
                    dtv = dt_vs[h][pl.ds(off, 16)]
                    relc = jnp.clip(sg - base, 0, BLK_SEG - 1)
                    tv = plsc.load_gather(times_v, [relc])
                    delta = tv - dtv
                    gi = cs + off + iota
                    valid = (gi >= lo_c) & (gi < hi_c)
                    idx = jnp.where(valid, relc * K + kd, DUMP)
                    idx_vs[h][pl.ds(off, 16)] = idx
                    for i in range(16):
                        d_s = delta[i]
                        e_lo = jnp.exp(d_s * nrate_lo)
                        e_hi = jnp.exp(d_s * nrate_hi)
                        r = off + i
                        vals_vs[h][r, pl.ds(0, 16)] = (
                            feat_vs[h][r, pl.ds(0, 16)] * e_lo)
                        vals_vs[h][r, pl.ds(16, 16)] = (
                            feat_vs[h][r, pl.ds(16, 16)] * e_hi)
                scats.append(pltpu.async_copy(
                    vals_vs[h], shared.at[idx_vs[h]], sem, add=True))
            for sc in scats:
                sc.wait()
            return carry2

        lax.fori_loop(0, nc2, pair_body, 0)
        plsc.subcore_barrier()

        def drain_body(q2, cd):
            r0 = (t + q2 * TILES) * CH_R
            pltpu.sync_copy(shared.at[pl.ds(r0, CH_R)],
                            out_hbm.at[pl.ds(row0 + r0, CH_R)])
            return cd

        lax.fori_loop(0, nq, drain_body, 0)
        return carry

    lax.fori_loop(0, BLK_PER_CORE, block_body, 0)


def kernel(features, dt, times_out, decay_rate, successor_kernel_ids,
           segment_ids_out):
    nrate = -jax.nn.softplus(decay_rate).astype(jnp.float32)
    starts = (jnp.arange(NB + 1, dtype=jnp.int32) * BLK_SEG)
    bounds = jnp.searchsorted(segment_ids_out, starts,
                              method="compare_all").astype(jnp.int32)
    bounds48 = jnp.concatenate(
        [bounds, jnp.full((48 - (NB + 1),), N_IN, dtype=jnp.int32)])
    zeros_c = jnp.zeros((CH_R, F), dtype=jnp.float32)

    kern = pl.kernel(
        _sc_body,
        out_type=jax.ShapeDtypeStruct((N_OUT * K, F), jnp.float32),
        mesh=plsc.VectorSubcoreMesh(core_axis_name="c", subcore_axis_name="s"),
        scratch_types=[
            pltpu.VMEM_SHARED((SH_ROWS, F), jnp.float32),
            pltpu.VMEM((BLK_SEG,), jnp.float32),
            pltpu.VMEM((CHUNK, F), jnp.float32),
            pltpu.VMEM((CHUNK,), jnp.float32),
            pltpu.VMEM((CHUNK,), jnp.int32),
            pltpu.VMEM((CHUNK,), jnp.int32),
            pltpu.VMEM((CHUNK, F), jnp.float32),
            pltpu.VMEM((CHUNK,), jnp.int32),
            pltpu.VMEM((CHUNK, F), jnp.float32),
            pltpu.VMEM((CHUNK,), jnp.float32),
            pltpu.VMEM((CHUNK,), jnp.int32),
            pltpu.VMEM((CHUNK,), jnp.int32),
            pltpu.VMEM((CHUNK, F), jnp.float32),
            pltpu.VMEM((CHUNK,), jnp.int32),
            pltpu.VMEM((48,), jnp.int32),
            pltpu.VMEM((F,), jnp.float32),
            pltpu.VMEM((CH_R, F), jnp.float32),
            pltpu.SemaphoreType.DMA,
        ],
        compiler_params=pltpu.CompilerParams(
            needs_layout_passes=False, use_tc_tiling_on_sc=False),
    )
    out2d = kern(features, dt, times_out, nrate, successor_kernel_ids,
                 segment_ids_out, bounds48, zeros_c)
    return out2d.reshape(N_OUT, K, F)
